# baseline (speedup 1.0000x reference)
import functools

import jax
import jax.numpy as jnp
from jax import lax
from jax.experimental import pallas as pl
from jax.experimental.pallas import tpu as pltpu
from jax.experimental.pallas import tpu_sc as plsc

N = 10000
E = 320000
D = 128
NP = 10240
NS = 16
NW = 2 * NS
EPT = E // NW
CH = 88
NCHUNK = 116
CEPT = NCHUNK * CH
NRING = 4
RPT = NP // NS
BLK = 512
NB = NP // BLK

_mesh = plsc.VectorSubcoreMesh(core_axis_name="c", subcore_axis_name="s")


@functools.partial(
    pl.kernel,
    out_type=jax.ShapeDtypeStruct((2 * NP, D), jnp.float32),
    mesh=_mesh,
    scratch_types=(
        [pltpu.VMEM((CH,), jnp.int32) for _ in range(NRING)]
        + [pltpu.VMEM((CH,), jnp.int32) for _ in range(NRING)]
        + [pltpu.VMEM((CH, D), jnp.float32) for _ in range(NRING)]
        + [pltpu.VMEM_SHARED((NP, D), jnp.float32)]
        + [pltpu.SemaphoreType.DMA for _ in range(2 * NRING)]
    ),
)
def _segsum_sc(y_hbm, srcp_hbm, dstp_hbm, out_hbm, *refs):
    src_v = refs[0:NRING]
    dst_v = refs[NRING:2 * NRING]
    rows = refs[2 * NRING:3 * NRING]
    acc_sh = refs[3 * NRING]
    gsem = refs[3 * NRING + 1:3 * NRING + 1 + NRING]
    isem = refs[3 * NRING + 1 + NRING:3 * NRING + 1 + 2 * NRING]

    c = lax.axis_index("c")
    s = lax.axis_index("s")
    r0 = s * RPT
    ebase = (c * NS + s) * CEPT

    for b in range(NRING):
        pltpu.async_copy(srcp_hbm.at[pl.ds(ebase + b * CH, CH)], src_v[b], isem[b])
        pltpu.async_copy(dstp_hbm.at[pl.ds(ebase + b * CH, CH)], dst_v[b], isem[b])
    pltpu.sync_copy(y_hbm.at[pl.ds(r0, RPT)], acc_sh.at[pl.ds(r0, RPT)])
    plsc.subcore_barrier()

    def body(i, carry):
        k0 = NRING * i
        for b in range(NRING):
            pltpu.make_async_copy(
                srcp_hbm.at[pl.ds(ebase, CH)], src_v[b], isem[b]).wait()
            pltpu.make_async_copy(
                dstp_hbm.at[pl.ds(ebase, CH)], dst_v[b], isem[b]).wait()
            pltpu.async_copy(y_hbm.at[src_v[b]], rows[b], gsem[b])
        for b in range(NRING):
            k = k0 + b
            pltpu.make_async_copy(y_hbm.at[src_v[b]], rows[b], gsem[b]).wait()
            pltpu.sync_copy(rows[b], acc_sh.at[dst_v[b]], add=True)

            @pl.when(k + NRING < NCHUNK)
            def _():
                off = ebase + (k + NRING) * CH
                pltpu.async_copy(srcp_hbm.at[pl.ds(off, CH)], src_v[b], isem[b])
                pltpu.async_copy(dstp_hbm.at[pl.ds(off, CH)], dst_v[b], isem[b])
        return carry

    lax.fori_loop(0, NCHUNK // NRING, body, 0)
    plsc.subcore_barrier()
    pltpu.sync_copy(acc_sh.at[pl.ds(r0, RPT)], out_hbm.at[pl.ds(c * NP + r0, RPT)])


def _mm_body(x_ref, w_ref, o_ref):
    o_ref[...] = lax.dot_general(
        x_ref[...], w_ref[...], (((1,), (1,)), ((), ())),
        preferred_element_type=jnp.float32)


def _relu_mm_body(sa_ref, sb_ref, y_ref, b_ref, w_ref, o_ref):
    h = jnp.maximum(sa_ref[...] + sb_ref[...] - y_ref[...] + b_ref[...], 0.0)
    o_ref[...] = lax.dot_general(
        h, w_ref[...], (((1,), (1,)), ((), ())),
        preferred_element_type=jnp.float32)


def _final_body(sa_ref, sb_ref, y_ref, b_ref, o_ref):
    o_ref[...] = sa_ref[...] + sb_ref[...] - y_ref[...] + b_ref[...]


def kernel(x, adj, W1, b1, W2, b2):
    srcp = jnp.pad(adj[0].reshape(NW, EPT),
                   ((0, 0), (0, CEPT - EPT))).reshape(-1)
    dstp = jnp.pad(adj[1].reshape(NW, EPT), ((0, 0), (0, CEPT - EPT)),
                   constant_values=NP - 1).reshape(-1)
    x_pad = jnp.pad(x, ((0, NP - N), (0, 0)))

    y1 = pl.pallas_call(
        _mm_body,
        grid=(NB,),
        in_specs=[
            pl.BlockSpec((BLK, D), lambda j: (j, 0)),
            pl.BlockSpec((D, D), lambda j: (0, 0)),
        ],
        out_specs=pl.BlockSpec((BLK, D), lambda j: (j, 0)),
        out_shape=jax.ShapeDtypeStruct((NP, D), jnp.float32),
    )(x_pad, W1)

    s1 = _segsum_sc(y1, srcp, dstp)

    y2 = pl.pallas_call(
        _relu_mm_body,
        grid=(NB,),
        in_specs=[
            pl.BlockSpec((BLK, D), lambda j: (j, 0)),
            pl.BlockSpec((BLK, D), lambda j: (NB + j, 0)),
            pl.BlockSpec((BLK, D), lambda j: (j, 0)),
            pl.BlockSpec((1, D), lambda j: (0, 0)),
            pl.BlockSpec((D, D), lambda j: (0, 0)),
        ],
        out_specs=pl.BlockSpec((BLK, D), lambda j: (j, 0)),
        out_shape=jax.ShapeDtypeStruct((NP, D), jnp.float32),
    )(s1, s1, y1, b1.reshape(1, D), W2)

    s2 = _segsum_sc(y2, srcp, dstp)

    out = pl.pallas_call(
        _final_body,
        grid=(NB,),
        in_specs=[
            pl.BlockSpec((BLK, D), lambda j: (j, 0)),
            pl.BlockSpec((BLK, D), lambda j: (NB + j, 0)),
            pl.BlockSpec((BLK, D), lambda j: (j, 0)),
            pl.BlockSpec((1, D), lambda j: (0, 0)),
        ],
        out_specs=pl.BlockSpec((BLK, D), lambda j: (j, 0)),
        out_shape=jax.ShapeDtypeStruct((NP, D), jnp.float32),
    )(s2, s2, y2, b2.reshape(1, D))

    return out[:N]


# speedup vs baseline: 1.0959x; 1.0959x over previous
import functools

import jax
import jax.numpy as jnp
from jax import lax
from jax.experimental import pallas as pl
from jax.experimental.pallas import tpu as pltpu
from jax.experimental.pallas import tpu_sc as plsc

N = 10000
E = 320000
D = 128
NP = 10240
NS = 16
NW = 2 * NS
EPT = E // NW
CH = 88
NCHUNK = 116
CEPT = NCHUNK * CH
NRING = 4
RPT = NP // NS
BLK = 512
NB = NP // BLK

_mesh = plsc.VectorSubcoreMesh(core_axis_name="c", subcore_axis_name="s")


@functools.partial(
    pl.kernel,
    out_type=jax.ShapeDtypeStruct((2 * NP, D), jnp.float32),
    mesh=_mesh,
    scratch_types=(
        [pltpu.VMEM((CH,), jnp.int32) for _ in range(NRING)]
        + [pltpu.VMEM((CH,), jnp.int32) for _ in range(NRING)]
        + [pltpu.VMEM((CH, D), jnp.float32) for _ in range(NRING)]
        + [pltpu.VMEM_SHARED((NP, D), jnp.float32)]
        + [pltpu.SemaphoreType.DMA for _ in range(4 * NRING)]
    ),
)
def _segsum_sc(y_hbm, srcp_hbm, dstp_hbm, out_hbm, *refs):
    src_v = refs[0:NRING]
    dst_v = refs[NRING:2 * NRING]
    rows = refs[2 * NRING:3 * NRING]
    acc_sh = refs[3 * NRING]
    sems = refs[3 * NRING + 1:]
    gsem = sems[0:NRING]
    csem = sems[NRING:2 * NRING]
    ssem = sems[2 * NRING:3 * NRING]
    dsem = sems[3 * NRING:4 * NRING]

    c = lax.axis_index("c")
    s = lax.axis_index("s")
    r0 = s * RPT
    ebase = (c * NS + s) * CEPT

    def fire_src(k, b):
        pltpu.async_copy(srcp_hbm.at[pl.ds(ebase + k * CH, CH)], src_v[b], ssem[b])

    def fire_dst(k, b):
        pltpu.async_copy(dstp_hbm.at[pl.ds(ebase + k * CH, CH)], dst_v[b], dsem[b])

    def wait_src(b):
        pltpu.make_async_copy(srcp_hbm.at[pl.ds(ebase, CH)], src_v[b], ssem[b]).wait()

    def wait_dst(b):
        pltpu.make_async_copy(dstp_hbm.at[pl.ds(ebase, CH)], dst_v[b], dsem[b]).wait()

    def fire_gather(b):
        pltpu.async_copy(y_hbm.at[src_v[b]], rows[b], gsem[b])

    def wait_gather(b):
        pltpu.make_async_copy(y_hbm.at[src_v[b]], rows[b], gsem[b]).wait()

    def fire_scatter(b):
        pltpu.async_copy(rows[b], acc_sh.at[dst_v[b]], csem[b], add=True)

    def wait_scatter(b):
        pltpu.make_async_copy(rows[b], acc_sh.at[dst_v[b]], csem[b]).wait()

    for b in range(NRING):
        fire_src(b, b)
    fire_dst(0, 0)
    fire_dst(1, 1)
    pltpu.sync_copy(y_hbm.at[pl.ds(r0, RPT)], acc_sh.at[pl.ds(r0, RPT)])
    plsc.subcore_barrier()
    wait_src(0)
    fire_gather(0)
    wait_src(1)
    fire_gather(1)

    def body(i, carry):
        k0 = NRING * i
        for b in range(NRING):
            k = k0 + b
            b2 = (b + 2) % NRING

            @pl.when(k >= 2)
            def _():
                wait_scatter(b2)

            @pl.when(k + 2 < NCHUNK)
            def _():
                fire_dst(k + 2, b2)
                wait_src(b2)
                fire_gather(b2)

            wait_gather(b)
            @pl.when(k + NRING < NCHUNK)
            def _():
                fire_src(k + NRING, b)
            wait_dst(b)
            fire_scatter(b)
        return carry

    lax.fori_loop(0, NCHUNK // NRING, body, 0)
    wait_scatter(2)
    wait_scatter(3)
    plsc.subcore_barrier()
    pltpu.sync_copy(acc_sh.at[pl.ds(r0, RPT)], out_hbm.at[pl.ds(c * NP + r0, RPT)])


def _mm_body(x_ref, w_ref, o_ref):
    o_ref[...] = lax.dot_general(
        x_ref[...], w_ref[...], (((1,), (1,)), ((), ())),
        preferred_element_type=jnp.float32)


def _relu_mm_body(sa_ref, sb_ref, y_ref, b_ref, w_ref, o_ref):
    h = jnp.maximum(sa_ref[...] + sb_ref[...] - y_ref[...] + b_ref[...], 0.0)
    o_ref[...] = lax.dot_general(
        h, w_ref[...], (((1,), (1,)), ((), ())),
        preferred_element_type=jnp.float32)


def _final_body(sa_ref, sb_ref, y_ref, b_ref, o_ref):
    o_ref[...] = sa_ref[...] + sb_ref[...] - y_ref[...] + b_ref[...]


def kernel(x, adj, W1, b1, W2, b2):
    srcp = jnp.pad(adj[0].reshape(NW, EPT),
                   ((0, 0), (0, CEPT - EPT))).reshape(-1)
    dstp = jnp.pad(adj[1].reshape(NW, EPT), ((0, 0), (0, CEPT - EPT)),
                   constant_values=NP - 1).reshape(-1)
    x_pad = jnp.pad(x, ((0, NP - N), (0, 0)))

    y1 = pl.pallas_call(
        _mm_body,
        grid=(NB,),
        in_specs=[
            pl.BlockSpec((BLK, D), lambda j: (j, 0)),
            pl.BlockSpec((D, D), lambda j: (0, 0)),
        ],
        out_specs=pl.BlockSpec((BLK, D), lambda j: (j, 0)),
        out_shape=jax.ShapeDtypeStruct((NP, D), jnp.float32),
    )(x_pad, W1)

    s1 = _segsum_sc(y1, srcp, dstp)

    y2 = pl.pallas_call(
        _relu_mm_body,
        grid=(NB,),
        in_specs=[
            pl.BlockSpec((BLK, D), lambda j: (j, 0)),
            pl.BlockSpec((BLK, D), lambda j: (NB + j, 0)),
            pl.BlockSpec((BLK, D), lambda j: (j, 0)),
            pl.BlockSpec((1, D), lambda j: (0, 0)),
            pl.BlockSpec((D, D), lambda j: (0, 0)),
        ],
        out_specs=pl.BlockSpec((BLK, D), lambda j: (j, 0)),
        out_shape=jax.ShapeDtypeStruct((NP, D), jnp.float32),
    )(s1, s1, y1, b1.reshape(1, D), W2)

    s2 = _segsum_sc(y2, srcp, dstp)

    out = pl.pallas_call(
        _final_body,
        grid=(NB,),
        in_specs=[
            pl.BlockSpec((BLK, D), lambda j: (j, 0)),
            pl.BlockSpec((BLK, D), lambda j: (NB + j, 0)),
            pl.BlockSpec((BLK, D), lambda j: (j, 0)),
            pl.BlockSpec((1, D), lambda j: (0, 0)),
        ],
        out_specs=pl.BlockSpec((BLK, D), lambda j: (j, 0)),
        out_shape=jax.ShapeDtypeStruct((NP, D), jnp.float32),
    )(s2, s2, y2, b2.reshape(1, D))

    return out[:N]
